# Initial kernel scaffold; baseline (speedup 1.0000x reference)
import jax
import jax.numpy as jnp
from jax.experimental import pallas as pl


def kernel(h, indices):
    raise NotImplementedError("write your pallas kernel here")


# speedup vs baseline: 4.8439x; 4.8439x over previous
import functools

import jax
import jax.numpy as jnp
from jax.experimental import pallas as pl
from jax.experimental.pallas import tpu as pltpu

NZ = 1024
B_BLK = 1024


def _permute_body(idx_ref, h_ref, o_ref, p_ref):
    @pl.when(pl.program_id(0) == 0)
    def _build_onehot():
        rows = jax.lax.broadcasted_iota(jnp.int32, (NZ, NZ), 0)
        p_ref[...] = (rows == idx_ref[...][None, :]).astype(jnp.float32)

    o_ref[...] = jnp.dot(h_ref[...], p_ref[...],
                         preferred_element_type=jnp.float32)


def kernel(h, indices):
    B, nz = h.shape
    grid = (B // B_BLK,)
    return pl.pallas_call(
        _permute_body,
        grid=grid,
        in_specs=[
            pl.BlockSpec((nz,), lambda b: (0,)),
            pl.BlockSpec((B_BLK, nz), lambda b: (b, 0)),
        ],
        out_specs=pl.BlockSpec((B_BLK, nz), lambda b: (b, 0)),
        out_shape=jax.ShapeDtypeStruct((B, nz), h.dtype),
        scratch_shapes=[pltpu.VMEM((NZ, NZ), jnp.float32)],
    )(indices, h)
